# baseline (speedup 1.0000x reference)
import functools
import math

import jax
import jax.numpy as jnp
from jax import lax
from jax.experimental import pallas as pl
from jax.experimental.pallas import tpu as pltpu
from jax.experimental.pallas import tpu_sc as plsc

_UNITS = 1000000
_NEG = 8192
_BATCH = 4096
_DIM = 64
_BB = 512

_LOG_U1 = math.log(float(_UNITS) + 1.0)
_LOG_CONST = math.log(float(_NEG)) - math.log(_LOG_U1)

_IDXW = 128
_NW = 32
_SCH = _NEG // _NW // _IDXW
_TCH = _BATCH // _NW // _IDXW


def _half_ids(src, dst, chunks):
    for c in range(chunks):
        for j in range(_IDXW // 16):
            dst[c, pl.ds(j * 16, 16)] = lax.shift_right_logical(
                src[c, pl.ds(j * 16, 16)], 1)


def _sc_w_body(table2_hbm, sampled2_hbm, targets2_hbm, sw2_out, tw2_out,
               sidx, tidx, spidx, tpidx, srows, trows, sem):
    wid = lax.axis_index("s") * 2 + lax.axis_index("c")
    s_per_w = _SCH * _IDXW
    t_per_w = _TCH * _IDXW

    pltpu.sync_copy(sampled2_hbm.at[pl.ds(wid * _SCH, _SCH)], sidx)
    pltpu.sync_copy(targets2_hbm.at[pl.ds(wid * _TCH, _TCH)], tidx)
    _half_ids(sidx, spidx, _SCH)
    _half_ids(tidx, tpidx, _TCH)

    cps = []
    for c in range(_SCH):
        cps.append(pltpu.async_copy(
            table2_hbm.at[spidx.at[c]], srows.at[pl.ds(c * _IDXW, _IDXW)], sem))
    for c in range(_TCH):
        cps.append(pltpu.async_copy(
            table2_hbm.at[tpidx.at[c]], trows.at[pl.ds(c * _IDXW, _IDXW)], sem))
    for cp in cps:
        cp.wait()

    pltpu.sync_copy(srows, sw2_out.at[pl.ds(wid * s_per_w, s_per_w)])
    pltpu.sync_copy(trows, tw2_out.at[pl.ds(wid * t_per_w, t_per_w)])


def _sc_gather_w(table2, sampled2, targets2):
    mesh = plsc.VectorSubcoreMesh(core_axis_name="c", subcore_axis_name="s")
    fn = functools.partial(
        pl.kernel, mesh=mesh,
        out_type=[
            jax.ShapeDtypeStruct((_NEG, 2 * _DIM), jnp.float32),
            jax.ShapeDtypeStruct((_BATCH, 2 * _DIM), jnp.float32),
        ],
        scratch_types=[
            pltpu.VMEM((_SCH, _IDXW), jnp.int32),
            pltpu.VMEM((_TCH, _IDXW), jnp.int32),
            pltpu.VMEM((_SCH, _IDXW), jnp.int32),
            pltpu.VMEM((_TCH, _IDXW), jnp.int32),
            pltpu.VMEM((_SCH * _IDXW, 2 * _DIM), jnp.float32),
            pltpu.VMEM((_TCH * _IDXW, 2 * _DIM), jnp.float32),
            pltpu.SemaphoreType.DMA,
        ],
    )(_sc_w_body)
    return fn(table2, sampled2, targets2)


def _sc_b_body(bias_hbm, sampled2_hbm, targets2_hbm, sb_out, tb_out,
               sidx, tidx, sbias, tbias, sem):
    wid = lax.axis_index("s") * 2 + lax.axis_index("c")
    s_per_w = _SCH * _IDXW
    t_per_w = _TCH * _IDXW

    pltpu.sync_copy(sampled2_hbm.at[pl.ds(wid * _SCH, _SCH)], sidx)
    pltpu.sync_copy(targets2_hbm.at[pl.ds(wid * _TCH, _TCH)], tidx)

    cps = []
    for c in range(_SCH):
        cps.append(pltpu.async_copy(
            bias_hbm.at[sidx.at[c]], sbias.at[pl.ds(c * _IDXW, _IDXW)], sem))
    for c in range(_TCH):
        cps.append(pltpu.async_copy(
            bias_hbm.at[tidx.at[c]], tbias.at[pl.ds(c * _IDXW, _IDXW)], sem))
    for cp in cps:
        cp.wait()

    pltpu.sync_copy(sbias, sb_out.at[pl.ds(wid * s_per_w, s_per_w)])
    pltpu.sync_copy(tbias, tb_out.at[pl.ds(wid * t_per_w, t_per_w)])


def _sc_gather_b(bias, sampled2, targets2):
    mesh = plsc.VectorSubcoreMesh(core_axis_name="c", subcore_axis_name="s")
    fn = functools.partial(
        pl.kernel, mesh=mesh,
        compiler_params=pltpu.CompilerParams(use_tc_tiling_on_sc=False),
        out_type=[
            jax.ShapeDtypeStruct((_NEG,), jnp.float32),
            jax.ShapeDtypeStruct((_BATCH,), jnp.float32),
        ],
        scratch_types=[
            pltpu.VMEM((_SCH, _IDXW), jnp.int32),
            pltpu.VMEM((_TCH, _IDXW), jnp.int32),
            pltpu.VMEM((_SCH * _IDXW,), jnp.float32),
            pltpu.VMEM((_TCH * _IDXW,), jnp.float32),
            pltpu.SemaphoreType.DMA,
        ],
    )(_sc_b_body)
    return fn(bias, sampled2, targets2)


def _loss_body(x_ref, tw2_ref, tb_ref, tid_ref, sw2_ref, sb_ref, sid_ref,
               sidc_ref, out_ref):
    i = pl.program_id(0)
    x = x_ref[...]
    tw2 = tw2_ref[...]
    tb = tb_ref[...]
    tid = tid_ref[...]
    sw2 = sw2_ref[...]
    sb = sb_ref[...]
    sid = sid_ref[...]
    sidc = sidc_ref[...]

    tw = jnp.where(lax.rem(tid, 2) == 1, tw2[:, _DIM:], tw2[:, :_DIM])
    sw = jnp.where(lax.rem(sidc, 2) == 1, sw2[:, _DIM:], sw2[:, :_DIM])

    tidf = tid.astype(jnp.float32)
    log_np_t = jnp.log(jnp.log(tidf + 2.0) - jnp.log(tidf + 1.0)) + _LOG_CONST
    true_logits = (jnp.sum(x * tw, axis=1, keepdims=True) + tb - log_np_t)

    sidf = sid.astype(jnp.float32)
    log_np_s = jnp.log(jnp.log(sidf + 2.0) - jnp.log(sidf + 1.0)) + _LOG_CONST
    sl = lax.dot_general(x, sw, (((1,), (1,)), ((), ())),
                         preferred_element_type=jnp.float32)
    sl = sl + (sb - log_np_s)
    sl = jnp.where(tid == sid, sl - 1e9, sl)

    m = jnp.maximum(jnp.max(sl, axis=1, keepdims=True), true_logits)
    se = jnp.sum(jnp.exp(sl - m), axis=1, keepdims=True) + jnp.exp(true_logits - m)
    per_ex = jnp.log(se) + m - true_logits
    part = jnp.sum(per_ex) * (1.0 / _BATCH)

    @pl.when(i == 0)
    def _():
        out_ref[0, 0] = 0.0

    out_ref[0, 0] += part


def _loss(logits, tw2, tb2, tid2, sw2, sb2, sid2, sidc):
    return pl.pallas_call(
        _loss_body,
        grid=(_BATCH // _BB,),
        in_specs=[
            pl.BlockSpec((_BB, _DIM), lambda i: (i, 0)),
            pl.BlockSpec((_BB, 2 * _DIM), lambda i: (i, 0)),
            pl.BlockSpec((_BB, 1), lambda i: (i, 0)),
            pl.BlockSpec((_BB, 1), lambda i: (i, 0)),
            pl.BlockSpec((_NEG, 2 * _DIM), lambda i: (0, 0)),
            pl.BlockSpec((1, _NEG), lambda i: (0, 0)),
            pl.BlockSpec((1, _NEG), lambda i: (0, 0)),
            pl.BlockSpec((_NEG, 1), lambda i: (0, 0)),
        ],
        out_specs=pl.BlockSpec(memory_space=pltpu.SMEM),
        out_shape=jax.ShapeDtypeStruct((1, 1), jnp.float32),
    )(logits, tw2, tb2, tid2, sw2, sb2, sid2, sidc)


def kernel(logits, targets, kernel, bias, sampled):
    table2 = kernel.reshape(_UNITS // 2, 2 * _DIM)
    sampled2 = sampled.reshape(_NEG // _IDXW, _IDXW)
    targets2 = targets.reshape(_BATCH // _IDXW, _IDXW)
    sw2, tw2 = _sc_gather_w(table2, sampled2, targets2)
    sb, tb = _sc_gather_b(bias, sampled2, targets2)
    out = _loss(logits, tw2, tb.reshape(_BATCH, 1), targets.reshape(_BATCH, 1),
                sw2, sb.reshape(1, _NEG), sampled.reshape(1, _NEG),
                sampled.reshape(_NEG, 1))
    return out[0, 0]


# speedup vs baseline: 1.0644x; 1.0644x over previous
import functools
import math

import jax
import jax.numpy as jnp
from jax import lax
from jax.experimental import pallas as pl
from jax.experimental.pallas import tpu as pltpu
from jax.experimental.pallas import tpu_sc as plsc

_UNITS = 1000000
_NEG = 8192
_BATCH = 4096
_DIM = 64
_BB = 512

_LOG_U1 = math.log(float(_UNITS) + 1.0)
_LOG_CONST = math.log(float(_NEG)) - math.log(_LOG_U1)

_IDXW = 128
_NW = 32
_SCH = _NEG // _NW // _IDXW
_TCH = _BATCH // _NW // _IDXW


def _sc_gather_body(table_hbm, bias_hbm, sampled2_hbm, targets2_hbm,
                    sw_out, tw_out, sb_out, tb_out,
                    sidx, tidx, srows, trows, sbias, tbias, sem):
    wid = lax.axis_index("s") * 2 + lax.axis_index("c")
    s_per_w = _SCH * _IDXW
    t_per_w = _TCH * _IDXW

    pltpu.sync_copy(sampled2_hbm.at[pl.ds(wid * _SCH, _SCH)], sidx)
    pltpu.sync_copy(targets2_hbm.at[pl.ds(wid * _TCH, _TCH)], tidx)

    cps = []
    for c in range(_SCH):
        cps.append(pltpu.async_copy(
            table_hbm.at[sidx.at[c]], srows.at[pl.ds(c * _IDXW, _IDXW)], sem))
        cps.append(pltpu.async_copy(
            bias_hbm.at[sidx.at[c]], sbias.at[pl.ds(c * _IDXW, _IDXW)], sem))
    for c in range(_TCH):
        cps.append(pltpu.async_copy(
            table_hbm.at[tidx.at[c]], trows.at[pl.ds(c * _IDXW, _IDXW)], sem))
        cps.append(pltpu.async_copy(
            bias_hbm.at[tidx.at[c]], tbias.at[pl.ds(c * _IDXW, _IDXW)], sem))
    for cp in cps:
        cp.wait()

    pltpu.sync_copy(srows, sw_out.at[pl.ds(wid * s_per_w, s_per_w)])
    pltpu.sync_copy(trows, tw_out.at[pl.ds(wid * t_per_w, t_per_w)])
    pltpu.sync_copy(sbias, sb_out.at[pl.ds(wid * s_per_w, s_per_w)])
    pltpu.sync_copy(tbias, tb_out.at[pl.ds(wid * t_per_w, t_per_w)])


def _sc_gather(table, bias, sampled2, targets2):
    mesh = plsc.VectorSubcoreMesh(core_axis_name="c", subcore_axis_name="s")
    fn = functools.partial(
        pl.kernel, mesh=mesh,
        compiler_params=pltpu.CompilerParams(use_tc_tiling_on_sc=False),
        out_type=[
            jax.ShapeDtypeStruct((_NEG, _DIM), jnp.float32),
            jax.ShapeDtypeStruct((_BATCH, _DIM), jnp.float32),
            jax.ShapeDtypeStruct((_NEG,), jnp.float32),
            jax.ShapeDtypeStruct((_BATCH,), jnp.float32),
        ],
        scratch_types=[
            pltpu.VMEM((_SCH, _IDXW), jnp.int32),
            pltpu.VMEM((_TCH, _IDXW), jnp.int32),
            pltpu.VMEM((_SCH * _IDXW, _DIM), jnp.float32),
            pltpu.VMEM((_TCH * _IDXW, _DIM), jnp.float32),
            pltpu.VMEM((_SCH * _IDXW,), jnp.float32),
            pltpu.VMEM((_TCH * _IDXW,), jnp.float32),
            pltpu.SemaphoreType.DMA,
        ],
    )(_sc_gather_body)
    return fn(table, bias, sampled2, targets2)


def _loss_body(x_ref, tw_ref, tb_ref, tid_ref, sw_ref, sb_ref, sid_ref, out_ref):
    i = pl.program_id(0)
    x = x_ref[...]
    tw = tw_ref[...]
    tb = tb_ref[...]
    tid = tid_ref[...]
    sw = sw_ref[...]
    sb = sb_ref[...]
    sid = sid_ref[...]

    tidf = tid.astype(jnp.float32)
    log_np_t = jnp.log(jnp.log(tidf + 2.0) - jnp.log(tidf + 1.0)) + _LOG_CONST
    true_logits = (jnp.sum(x * tw, axis=1, keepdims=True) + tb - log_np_t)

    sidf = sid.astype(jnp.float32)
    log_np_s = jnp.log(jnp.log(sidf + 2.0) - jnp.log(sidf + 1.0)) + _LOG_CONST
    sl = lax.dot_general(x, sw, (((1,), (1,)), ((), ())),
                         preferred_element_type=jnp.float32)
    sl = sl + (sb - log_np_s)
    sl = jnp.where(tid == sid, sl - 1e9, sl)

    m = jnp.maximum(jnp.max(sl, axis=1, keepdims=True), true_logits)
    se = jnp.sum(jnp.exp(sl - m), axis=1, keepdims=True) + jnp.exp(true_logits - m)
    per_ex = jnp.log(se) + m - true_logits
    part = jnp.sum(per_ex) * (1.0 / _BATCH)

    @pl.when(i == 0)
    def _():
        out_ref[0, 0] = 0.0

    out_ref[0, 0] += part


def _loss(logits, tw, tb2, tid2, sw, sb2, sid2):
    return pl.pallas_call(
        _loss_body,
        grid=(_BATCH // _BB,),
        in_specs=[
            pl.BlockSpec((_BB, _DIM), lambda i: (i, 0)),
            pl.BlockSpec((_BB, _DIM), lambda i: (i, 0)),
            pl.BlockSpec((_BB, 1), lambda i: (i, 0)),
            pl.BlockSpec((_BB, 1), lambda i: (i, 0)),
            pl.BlockSpec((_NEG, _DIM), lambda i: (0, 0)),
            pl.BlockSpec((1, _NEG), lambda i: (0, 0)),
            pl.BlockSpec((1, _NEG), lambda i: (0, 0)),
        ],
        out_specs=pl.BlockSpec(memory_space=pltpu.SMEM),
        out_shape=jax.ShapeDtypeStruct((1, 1), jnp.float32),
    )(logits, tw, tb2, tid2, sw, sb2, sid2)


def kernel(logits, targets, kernel, bias, sampled):
    sampled2 = sampled.reshape(_NEG // _IDXW, _IDXW)
    targets2 = targets.reshape(_BATCH // _IDXW, _IDXW)
    sw, tw, sb, tb = _sc_gather(kernel, bias, sampled2, targets2)
    out = _loss(logits, tw, tb.reshape(_BATCH, 1), targets.reshape(_BATCH, 1),
                sw, sb.reshape(1, _NEG), sampled.reshape(1, _NEG))
    return out[0, 0]
